# baseline (speedup 1.0000x reference)
import dataclasses

import jax
import jax.numpy as jnp
from jax import lax
from jax.experimental import pallas as pl
from jax.experimental.compute_on import compute_on
from jax.experimental.pallas import tpu as pltpu
from jax.experimental.pallas import tpu_sc as plsc

_ROWS = 16384
_COLS = 1024
_LANES = 16

_SC_ROWS = 3072
_TC_ROWS = _ROWS - _SC_ROWS
_TC_BLK = 1024
_SC_BR = 16

_mesh = plsc.VectorSubcoreMesh(core_axis_name="c", subcore_axis_name="s")

_cparams = pltpu.CompilerParams()
if "needs_layout_passes" in pltpu.CompilerParams.__dataclass_fields__:
    _cparams = dataclasses.replace(_cparams, needs_layout_passes=False)


def _tc_body(idx_ref, b0_ref, b1_ref, b2_ref, x1_ref, x2_ref, o_ref):
    h = idx_ref[0]
    c0 = b0_ref[h]
    c1 = b1_ref[h]
    c2 = b2_ref[h]
    o_ref[:] = c0 + c1 * x1_ref[:] + c2 * x2_ref[:]


def _tc_part(idx, b0, b1, b2, x1, x2):
    row_off = _SC_ROWS // _TC_BLK
    return pl.pallas_call(
        _tc_body,
        grid=(_TC_ROWS // _TC_BLK,),
        in_specs=[
            pl.BlockSpec(memory_space=pltpu.SMEM),
            pl.BlockSpec(memory_space=pltpu.SMEM),
            pl.BlockSpec(memory_space=pltpu.SMEM),
            pl.BlockSpec(memory_space=pltpu.SMEM),
            pl.BlockSpec((_TC_BLK, _COLS), lambda i: (i + row_off, 0)),
            pl.BlockSpec((_TC_BLK, _COLS), lambda i: (i + row_off, 0)),
        ],
        out_specs=pl.BlockSpec((_TC_BLK, _COLS), lambda i: (i + row_off, 0)),
        out_shape=jax.ShapeDtypeStruct((_ROWS, _COLS), jnp.float32),
    )(idx, b0, b1, b2, x1, x2)


def _sc_body(idx_hbm, b0_hbm, b1_hbm, b2_hbm, x1_hbm, x2_hbm, o_hbm,
             idx_vmem, b0_vmem, b1_vmem, b2_vmem, sem):
    pltpu.async_copy(idx_hbm, idx_vmem, sem).wait()
    pltpu.async_copy(b0_hbm, b0_vmem, sem).wait()
    pltpu.async_copy(b1_hbm, b1_vmem, sem).wait()
    pltpu.async_copy(b2_hbm, b2_vmem, sem).wait()
    idxv = idx_vmem[...]
    c0 = plsc.load_gather(b0_vmem, [idxv])
    c1 = plsc.load_gather(b1_vmem, [idxv])
    c2 = plsc.load_gather(b2_vmem, [idxv])

    def block_body(x1_v, x2_v, o_v):
        @pl.loop(0, _SC_BR)
        def _(r):
            @plsc.parallel_loop(0, _COLS, step=_LANES, unroll=8)
            def _(c):
                v1 = x1_v.at[r, pl.ds(c, _LANES)][...]
                v2 = x2_v.at[r, pl.ds(c, _LANES)][...]
                o_v.at[r, pl.ds(c, _LANES)][...] = c0 + c1 * v1 + c2 * v2

    pltpu.emit_pipeline(
        block_body,
        grid=(_SC_ROWS // _SC_BR,),
        in_specs=[
            pl.BlockSpec((_SC_BR, _COLS), lambda i: (i, 0)),
            pl.BlockSpec((_SC_BR, _COLS), lambda i: (i, 0)),
        ],
        out_specs=[pl.BlockSpec((_SC_BR, _COLS), lambda i: (i, 0))],
        core_axis_name=("c", "s"),
        dimension_semantics=(pltpu.PARALLEL,),
    )(x1_hbm, x2_hbm, o_hbm)


def _sc_part(idxv, b0, b1, b2, x1, x2):
    k = pl.kernel(
        _sc_body,
        out_type=jax.ShapeDtypeStruct((_SC_ROWS, _COLS), jnp.float32),
        mesh=_mesh,
        scratch_types=[
            pltpu.VMEM((_LANES,), jnp.int32),
            pltpu.VMEM((168,), jnp.float32),
            pltpu.VMEM((168,), jnp.float32),
            pltpu.VMEM((168,), jnp.float32),
            pltpu.SemaphoreType.DMA,
        ],
        compiler_params=_cparams,
    )
    return k(idxv, b0, b1, b2, x1, x2)


def kernel(hour_idx, x1, x2, b0, b1, b2):
    idx = jnp.asarray(hour_idx, jnp.int32).reshape(1)
    idxv = jnp.full((_LANES,), hour_idx, jnp.int32)
    sc_out = compute_on("tpu_sparsecore")(_sc_part)(idxv, b0, b1, b2, x1, x2)
    tc_out = _tc_part(idx, b0, b1, b2, x1, x2)
    return lax.dynamic_update_slice(tc_out, sc_out, (0, 0))


# speedup vs baseline: 1.0009x; 1.0009x over previous
import dataclasses

import jax
import jax.numpy as jnp
from jax import lax
from jax.experimental import pallas as pl
from jax.experimental.compute_on import compute_on
from jax.experimental.pallas import tpu as pltpu
from jax.experimental.pallas import tpu_sc as plsc

_ROWS = 16384
_COLS = 1024
_LANES = 16

_SC_ROWS = 3072
_TC_ROWS = _ROWS - _SC_ROWS
_TC_BLK = 1024
_SC_BR = 16

_mesh = plsc.VectorSubcoreMesh(core_axis_name="c", subcore_axis_name="s")

_cparams = pltpu.CompilerParams()
if "needs_layout_passes" in pltpu.CompilerParams.__dataclass_fields__:
    _cparams = dataclasses.replace(_cparams, needs_layout_passes=False)


def _tc_body(idx_ref, b0_ref, b1_ref, b2_ref, x1_ref, x2_ref, o_ref):
    h = idx_ref[0]
    c0 = b0_ref[h]
    c1 = b1_ref[h]
    c2 = b2_ref[h]
    o_ref[:] = c0 + c1 * x1_ref[:] + c2 * x2_ref[:]


def _tc_part(idx, b0, b1, b2, x1, x2):
    row_off = _SC_ROWS // _TC_BLK
    return pl.pallas_call(
        _tc_body,
        grid=(_TC_ROWS // _TC_BLK,),
        in_specs=[
            pl.BlockSpec(memory_space=pltpu.SMEM),
            pl.BlockSpec(memory_space=pltpu.SMEM),
            pl.BlockSpec(memory_space=pltpu.SMEM),
            pl.BlockSpec(memory_space=pltpu.SMEM),
            pl.BlockSpec((_TC_BLK, _COLS), lambda i: (i + row_off, 0)),
            pl.BlockSpec((_TC_BLK, _COLS), lambda i: (i + row_off, 0)),
        ],
        out_specs=pl.BlockSpec((_TC_BLK, _COLS), lambda i: (i + row_off, 0)),
        out_shape=jax.ShapeDtypeStruct((_ROWS, _COLS), jnp.float32),
    )(idx, b0, b1, b2, x1, x2)


def _sc_body(idx_hbm, b0_hbm, b1_hbm, b2_hbm, x1_hbm, x2_hbm, o_hbm,
             idx_vmem, b0_vmem, b1_vmem, b2_vmem, sem):
    pltpu.async_copy(idx_hbm, idx_vmem, sem).wait()
    pltpu.async_copy(b0_hbm, b0_vmem, sem).wait()
    pltpu.async_copy(b1_hbm, b1_vmem, sem).wait()
    pltpu.async_copy(b2_hbm, b2_vmem, sem).wait()
    idxv = idx_vmem[...]
    c0 = plsc.load_gather(b0_vmem, [idxv])
    c1 = plsc.load_gather(b1_vmem, [idxv])
    c2 = plsc.load_gather(b2_vmem, [idxv])

    def block_body(x1_v, x2_v, o_v):
        @pl.loop(0, _SC_BR)
        def _(r):
            @plsc.parallel_loop(0, _COLS, step=_LANES, unroll=8)
            def _(c):
                v1 = x1_v.at[r, pl.ds(c, _LANES)][...]
                v2 = x2_v.at[r, pl.ds(c, _LANES)][...]
                o_v.at[r, pl.ds(c, _LANES)][...] = c0 + c1 * v1 + c2 * v2

    pltpu.emit_pipeline(
        block_body,
        grid=(_SC_ROWS // _SC_BR,),
        in_specs=[
            pl.BlockSpec((_SC_BR, _COLS), lambda i: (i, 0)),
            pl.BlockSpec((_SC_BR, _COLS), lambda i: (i, 0)),
        ],
        out_specs=[pl.BlockSpec((_SC_BR, _COLS), lambda i: (i, 0))],
        core_axis_name=("c", "s"),
        dimension_semantics=(pltpu.PARALLEL,),
    )(x1_hbm, x2_hbm, o_hbm)


def _sc_part(idxv, b0, b1, b2, x1, x2):
    k = pl.kernel(
        _sc_body,
        out_type=jax.ShapeDtypeStruct((_SC_ROWS, _COLS), jnp.float32),
        mesh=_mesh,
        scratch_types=[
            pltpu.VMEM((_LANES,), jnp.int32),
            pltpu.VMEM((168,), jnp.float32),
            pltpu.VMEM((168,), jnp.float32),
            pltpu.VMEM((168,), jnp.float32),
            pltpu.SemaphoreType.DMA,
        ],
        compiler_params=_cparams,
    )
    return k(idxv, b0, b1, b2, x1, x2)


def kernel(hour_idx, x1, x2, b0, b1, b2):
    idx = jnp.asarray(hour_idx, jnp.int32).reshape(1)
    idxv = jnp.full((_LANES,), hour_idx, jnp.int32)
    tc_out = _tc_part(idx, b0, b1, b2, x1, x2)
    sc_out = compute_on("tpu_sparsecore")(_sc_part)(idxv, b0, b1, b2, x1, x2)
    return lax.dynamic_update_slice(tc_out, sc_out, (0, 0))


# speedup vs baseline: 1.4524x; 1.4511x over previous
import jax
import jax.numpy as jnp
from jax.experimental import pallas as pl
from jax.experimental.pallas import tpu as pltpu

_ROWS = 16384
_COLS = 1024
_CH = 512
_NCHUNK = _ROWS // _CH
_NB = 4


def _body(idx_ref, b0_ref, b1_ref, b2_ref, x1_hbm, x2_hbm, o_hbm,
          x1b, x2b, ob, s1, s2, so):
    h = idx_ref[0]
    c0 = b0_ref[h]
    c1 = b1_ref[h]
    c2 = b2_ref[h]

    def in_copy(i, s):
        r = pl.ds(i * _CH, _CH)
        return (pltpu.make_async_copy(x1_hbm.at[r], x1b.at[s], s1.at[s]),
                pltpu.make_async_copy(x2_hbm.at[r], x2b.at[s], s2.at[s]))

    def out_copy(i, s):
        r = pl.ds(i * _CH, _CH)
        return pltpu.make_async_copy(ob.at[s], o_hbm.at[r], so.at[s])

    for b in range(_NB):
        a, c = in_copy(b, b)
        a.start()
        c.start()

    for i in range(_NCHUNK):
        s = i % _NB
        a, c = in_copy(i, s)
        a.wait()
        c.wait()
        if i >= _NB:
            out_copy(i - _NB, s).wait()
        ob[s] = c0 + c1 * x1b[s] + c2 * x2b[s]
        out_copy(i, s).start()
        if i + _NB < _NCHUNK:
            a, c = in_copy(i + _NB, s)
            a.start()
            c.start()

    for i in range(_NCHUNK - _NB, _NCHUNK):
        out_copy(i, i % _NB).wait()


def kernel(hour_idx, x1, x2, b0, b1, b2):
    idx = jnp.asarray(hour_idx, jnp.int32).reshape(1)
    out = pl.pallas_call(
        _body,
        in_specs=[
            pl.BlockSpec(memory_space=pltpu.MemorySpace.SMEM),
            pl.BlockSpec(memory_space=pltpu.MemorySpace.SMEM),
            pl.BlockSpec(memory_space=pltpu.MemorySpace.SMEM),
            pl.BlockSpec(memory_space=pltpu.MemorySpace.SMEM),
            pl.BlockSpec(memory_space=pltpu.MemorySpace.HBM),
            pl.BlockSpec(memory_space=pltpu.MemorySpace.HBM),
        ],
        out_specs=pl.BlockSpec(memory_space=pltpu.MemorySpace.HBM),
        out_shape=jax.ShapeDtypeStruct((_ROWS, _COLS), jnp.float32),
        scratch_shapes=[
            pltpu.VMEM((_NB, _CH, _COLS), jnp.float32),
            pltpu.VMEM((_NB, _CH, _COLS), jnp.float32),
            pltpu.VMEM((_NB, _CH, _COLS), jnp.float32),
            pltpu.SemaphoreType.DMA((_NB,)),
            pltpu.SemaphoreType.DMA((_NB,)),
            pltpu.SemaphoreType.DMA((_NB,)),
        ],
    )(idx, b0, b1, b2, x1, x2)
    return out


# speedup vs baseline: 1.4595x; 1.0049x over previous
import jax
import jax.numpy as jnp
from jax.experimental import pallas as pl
from jax.experimental.pallas import tpu as pltpu

_ROWS = 16384
_COLS = 1024
_MAXCH = 1024
_NB = 4

_CHUNKS = []
_off = 0
for _n in [256] * 4 + [1024] * 14 + [256] * 4:
    _CHUNKS.append((_off, _n))
    _off += _n
assert _off == _ROWS


def _body(idx_ref, b0_ref, b1_ref, b2_ref, x1_hbm, x2_hbm, o_hbm,
          x1b, x2b, ob, s1, s2, so):
    h = idx_ref[0]
    c0 = b0_ref[h]
    c1 = b1_ref[h]
    c2 = b2_ref[h]

    def in_copy(i, s):
        off, n = _CHUNKS[i]
        r = pl.ds(off, n)
        return (pltpu.make_async_copy(x1_hbm.at[r], x1b.at[s, pl.ds(0, n)], s1.at[s]),
                pltpu.make_async_copy(x2_hbm.at[r], x2b.at[s, pl.ds(0, n)], s2.at[s]))

    def out_copy(i, s):
        off, n = _CHUNKS[i]
        return pltpu.make_async_copy(ob.at[s, pl.ds(0, n)],
                                     o_hbm.at[pl.ds(off, n)], so.at[s])

    for b in range(_NB):
        a, c = in_copy(b, b)
        a.start()
        c.start()

    for i in range(len(_CHUNKS)):
        s = i % _NB
        n = _CHUNKS[i][1]
        a, c = in_copy(i, s)
        a.wait()
        c.wait()
        if i >= _NB:
            out_copy(i - _NB, s).wait()
        ob[s, :n] = c0 + c1 * x1b[s, :n] + c2 * x2b[s, :n]
        out_copy(i, s).start()
        if i + _NB < len(_CHUNKS):
            a, c = in_copy(i + _NB, s)
            a.start()
            c.start()

    for i in range(len(_CHUNKS) - _NB, len(_CHUNKS)):
        out_copy(i, i % _NB).wait()


def kernel(hour_idx, x1, x2, b0, b1, b2):
    idx = jnp.asarray(hour_idx, jnp.int32).reshape(1)
    out = pl.pallas_call(
        _body,
        in_specs=[
            pl.BlockSpec(memory_space=pltpu.MemorySpace.SMEM),
            pl.BlockSpec(memory_space=pltpu.MemorySpace.SMEM),
            pl.BlockSpec(memory_space=pltpu.MemorySpace.SMEM),
            pl.BlockSpec(memory_space=pltpu.MemorySpace.SMEM),
            pl.BlockSpec(memory_space=pltpu.MemorySpace.HBM),
            pl.BlockSpec(memory_space=pltpu.MemorySpace.HBM),
        ],
        out_specs=pl.BlockSpec(memory_space=pltpu.MemorySpace.HBM),
        out_shape=jax.ShapeDtypeStruct((_ROWS, _COLS), jnp.float32),
        scratch_shapes=[
            pltpu.VMEM((_NB, _MAXCH, _COLS), jnp.float32),
            pltpu.VMEM((_NB, _MAXCH, _COLS), jnp.float32),
            pltpu.VMEM((_NB, _MAXCH, _COLS), jnp.float32),
            pltpu.SemaphoreType.DMA((_NB,)),
            pltpu.SemaphoreType.DMA((_NB,)),
            pltpu.SemaphoreType.DMA((_NB,)),
        ],
    )(idx, b0, b1, b2, x1, x2)
    return out


# speedup vs baseline: 1.4682x; 1.0060x over previous
import jax
import jax.numpy as jnp
from jax.experimental import pallas as pl
from jax.experimental.pallas import tpu as pltpu

_ROWS = 16384
_COLS = 1024
_BLK = 1024


def _body(idx_ref, b0_ref, b1_ref, b2_ref, x1_ref, x2_ref, o_ref):
    h = idx_ref[0]
    c0 = b0_ref[h]
    c1 = b1_ref[h]
    c2 = b2_ref[h]
    o_ref[:] = c0 + c1 * x1_ref[:] + c2 * x2_ref[:]


def kernel(hour_idx, x1, x2, b0, b1, b2):
    idx = jnp.asarray(hour_idx, jnp.int32).reshape(1)
    grid = (_ROWS // _BLK,)
    out = pl.pallas_call(
        _body,
        grid=grid,
        in_specs=[
            pl.BlockSpec(memory_space=pltpu.SMEM),
            pl.BlockSpec(memory_space=pltpu.SMEM),
            pl.BlockSpec(memory_space=pltpu.SMEM),
            pl.BlockSpec(memory_space=pltpu.SMEM),
            pl.BlockSpec((_BLK, _COLS), lambda i: (i, 0)),
            pl.BlockSpec((_BLK, _COLS), lambda i: (i, 0)),
        ],
        out_specs=pl.BlockSpec((_BLK, _COLS), lambda i: (i, 0)),
        out_shape=jax.ShapeDtypeStruct((_ROWS, _COLS), jnp.float32),
    )(idx, b0, b1, b2, x1, x2)
    return out
